# baseline (speedup 1.0000x reference)
import functools

import jax
import jax.numpy as jnp
from jax import lax
from jax.experimental import pallas as pl
from jax.experimental.pallas import tpu as pltpu
from jax.experimental.pallas import tpu_sc as plsc

R = 128
C = 32768
B = 512
NW = 32
NCH = 16
CHW = 128
LP = CHW // 16

_mesh = plsc.VectorSubcoreMesh(core_axis_name="c", subcore_axis_name="s")


@functools.partial(
    pl.kernel,
    out_type=jax.ShapeDtypeStruct((NW, 16), jnp.float32),
    mesh=_mesh,
    scratch_types=[
        pltpu.VMEM((NCH, CHW), jnp.int32),
        pltpu.VMEM((NCH, CHW), jnp.float32),
        pltpu.VMEM((NCH, CHW), jnp.float32),
        pltpu.VMEM((16,), jnp.float32),
        pltpu.SemaphoreType.DMA,
        pltpu.SemaphoreType.DMA,
    ],
)
def _l1_gather_kernel(out_flat, indb, tgtb, parts, ind_v, tgt_v,
                      gath_v, acc_v, insem, gsem):
    wid = lax.axis_index("s") * 2 + lax.axis_index("c")

    cp_tgt = pltpu.make_async_copy(tgtb.at[wid], tgt_v, insem)
    cp_tgt.start()
    pltpu.sync_copy(indb.at[wid], ind_v)

    row_base = (wid >> 1) << 18

    def fire_row(i, _):
        base = row_base + ((i & 7) << 7)
        for k in range(LP):
            c = ind_v[i, pl.ds(k * 16, 16)]
            ind_v[i, pl.ds(k * 16, 16)] = base + ((c >> 7) << 10) + (c & 127)
        pltpu.make_async_copy(
            out_flat.at[ind_v.at[i]], gath_v.at[i], gsem).start()
        return 0

    lax.fori_loop(0, NCH, fire_row, 0, unroll=False)

    cp_tgt.wait()

    def drain_row(i, accs):
        acc0, acc1 = accs
        pltpu.make_async_copy(
            out_flat.at[ind_v.at[i]], gath_v.at[i], gsem).wait()
        for k in range(LP):
            g = gath_v[i, pl.ds(k * 16, 16)]
            t = tgt_v[i, pl.ds(k * 16, 16)]
            if k % 2 == 0:
                acc0 = acc0 + jnp.abs(g - t)
            else:
                acc1 = acc1 + jnp.abs(g - t)
        return acc0, acc1

    z = jnp.zeros((16,), jnp.float32)
    acc0, acc1 = lax.fori_loop(0, NCH, drain_row, (z, z), unroll=False)

    acc_v[...] = acc0 + acc1
    pltpu.sync_copy(acc_v, parts.at[wid])


def _tile_major_flat(x):
    r, c = x.shape
    return x.reshape(r // 8, 8, c // 128, 128).transpose(0, 2, 1, 3).reshape(-1)


def kernel(output, ind, target):
    out_flat = _tile_major_flat(output)
    indb = _tile_major_flat(ind.astype(jnp.int32)).reshape(NW, NCH, CHW)
    tgtb = _tile_major_flat(target).reshape(NW, NCH, CHW)
    parts = _l1_gather_kernel(out_flat, indb, tgtb)
    return jnp.sum(parts)


# speedup vs baseline: 1.0058x; 1.0058x over previous
import functools

import jax
import jax.numpy as jnp
from jax import lax
from jax.experimental import pallas as pl
from jax.experimental.pallas import tpu as pltpu
from jax.experimental.pallas import tpu_sc as plsc

R = 128
C = 32768
B = 512
NW = 32
NCH = 16
CHW = 128
LP = CHW // 16

_mesh = plsc.VectorSubcoreMesh(core_axis_name="c", subcore_axis_name="s")


@functools.partial(
    pl.kernel,
    out_type=jax.ShapeDtypeStruct((NW, 16), jnp.float32),
    mesh=_mesh,
    scratch_types=[
        pltpu.VMEM((NCH, CHW), jnp.int32),
        pltpu.VMEM((NCH, CHW), jnp.float32),
        pltpu.VMEM((NCH, CHW), jnp.float32),
        pltpu.VMEM((16,), jnp.float32),
        pltpu.SemaphoreType.DMA,
        pltpu.SemaphoreType.DMA,
    ],
)
def _l1_gather_kernel(out_flat, indb, tgtb, parts, ind_v, tgt_v,
                      gath_v, acc_v, insem, gsem):
    wid = lax.axis_index("s") * 2 + lax.axis_index("c")

    with jax.named_scope("stage"):
        cp_tgt = pltpu.make_async_copy(tgtb.at[wid], tgt_v, insem)
        cp_tgt.start()
        pltpu.sync_copy(indb.at[wid], ind_v)

    row_base = (wid >> 1) << 18

    def fire_row(i, _):
        base = row_base + ((i & 7) << 7)
        for k in range(LP):
            c = ind_v[i, pl.ds(k * 16, 16)]
            ind_v[i, pl.ds(k * 16, 16)] = base + ((c >> 7) << 10) + (c & 127)
        pltpu.make_async_copy(
            out_flat.at[ind_v.at[i]], gath_v.at[i], gsem).start()
        return 0

    with jax.named_scope("fire"):
        lax.fori_loop(0, NCH, fire_row, 0, unroll=False)

    with jax.named_scope("tgtwait"):
        cp_tgt.wait()

    def drain_row(i, accs):
        acc0, acc1 = accs
        pltpu.make_async_copy(
            out_flat.at[ind_v.at[i]], gath_v.at[i], gsem).wait()
        for k in range(LP):
            g = gath_v[i, pl.ds(k * 16, 16)]
            t = tgt_v[i, pl.ds(k * 16, 16)]
            if k % 2 == 0:
                acc0 = acc0 + jnp.abs(g - t)
            else:
                acc1 = acc1 + jnp.abs(g - t)
        return acc0, acc1

    z = jnp.zeros((16,), jnp.float32)
    with jax.named_scope("drain"):
        acc0, acc1 = lax.fori_loop(0, NCH, drain_row, (z, z), unroll=False)

    with jax.named_scope("writeout"):
        acc_v[...] = acc0 + acc1
        pltpu.sync_copy(acc_v, parts.at[wid])


def _tile_major_flat(x):
    r, c = x.shape
    return x.reshape(r // 8, 8, c // 128, 128).transpose(0, 2, 1, 3).reshape(-1)


def kernel(output, ind, target):
    out_flat = _tile_major_flat(output)
    indb = _tile_major_flat(ind.astype(jnp.int32)).reshape(NW, NCH, CHW)
    tgtb = _tile_major_flat(target).reshape(NW, NCH, CHW)
    parts = _l1_gather_kernel(out_flat, indb, tgtb)
    return jnp.sum(parts)


# speedup vs baseline: 1.0114x; 1.0055x over previous
import functools

import jax
import jax.numpy as jnp
from jax import lax
from jax.experimental import pallas as pl
from jax.experimental.pallas import tpu as pltpu
from jax.experimental.pallas import tpu_sc as plsc

R = 128
C = 32768
B = 512
NW = 32
NCH = 16
CHW = 128
LP = CHW // 16

_mesh = plsc.VectorSubcoreMesh(core_axis_name="c", subcore_axis_name="s")


@functools.partial(
    pl.kernel,
    out_type=jax.ShapeDtypeStruct((NW * 16,), jnp.float32),
    mesh=_mesh,
    scratch_types=[
        pltpu.VMEM((NCH, CHW), jnp.int32),
        pltpu.VMEM((NCH, CHW), jnp.float32),
        pltpu.VMEM((NCH, CHW), jnp.float32),
        pltpu.VMEM((16,), jnp.float32),
        pltpu.SemaphoreType.DMA,
        pltpu.SemaphoreType.DMA,
    ],
)
def _l1_gather_kernel(out_flat, indb, tgtb, parts, ind_v, tgt_v,
                      gath_v, acc_v, insem, gsem):
    wid = lax.axis_index("s") * 2 + lax.axis_index("c")

    cp_tgt = pltpu.make_async_copy(tgtb.at[wid], tgt_v, insem)
    cp_tgt.start()
    pltpu.sync_copy(indb.at[wid], ind_v)

    row_base = (wid >> 1) << 18

    def fire_row(i, _):
        base = row_base + ((i & 7) << 7)
        for k in range(LP):
            c = ind_v[i, pl.ds(k * 16, 16)]
            ind_v[i, pl.ds(k * 16, 16)] = base + ((c >> 7) << 10) + (c & 127)
        pltpu.make_async_copy(
            out_flat.at[ind_v.at[i]], gath_v.at[i], gsem).start()
        return 0

    lax.fori_loop(0, NCH, fire_row, 0, unroll=False)

    cp_tgt.wait()

    def drain_row(i, accs):
        acc0, acc1 = accs
        pltpu.make_async_copy(
            out_flat.at[ind_v.at[i]], gath_v.at[i], gsem).wait()
        for k in range(LP):
            g = gath_v[i, pl.ds(k * 16, 16)]
            t = tgt_v[i, pl.ds(k * 16, 16)]
            if k % 2 == 0:
                acc0 = acc0 + jnp.abs(g - t)
            else:
                acc1 = acc1 + jnp.abs(g - t)
        return acc0, acc1

    z = jnp.zeros((16,), jnp.float32)
    acc0, acc1 = lax.fori_loop(0, NCH, drain_row, (z, z), unroll=False)

    acc_v[...] = acc0 + acc1
    pltpu.sync_copy(acc_v, parts.at[pl.ds(wid * 16, 16)])


def _tile_major_flat(x):
    r, c = x.shape
    return x.reshape(r // 8, 8, c // 128, 128).transpose(0, 2, 1, 3).reshape(-1)


def kernel(output, ind, target):
    out_flat = _tile_major_flat(output)
    indb = _tile_major_flat(ind.astype(jnp.int32)).reshape(NW, NCH, CHW)
    tgtb = _tile_major_flat(target).reshape(NW, NCH, CHW)
    parts = _l1_gather_kernel(out_flat, indb, tgtb)
    return jnp.sum(parts)
